# baseline (speedup 1.0000x reference)
import functools

import jax
import jax.numpy as jnp
from jax import lax
from jax.experimental import pallas as pl
from jax.experimental.pallas import tpu as pltpu
from jax.experimental.pallas import tpu_sc as plsc

_B, _S, _D, _K = 16, 576, 64, 1024
_CC = 0.99


def _row_sum(sq):
    acc = sq[:, 0:8]
    for r in range(1, 8):
        acc = acc + sq[:, 8 * r:8 * r + 8]
    for w in (4, 2, 1):
        acc = acc[:, :w] + acc[:, w:2 * w]
    return acc


_BASE_I = 0x3F700000


def _stage_a_body(inp_ref, emb_ref, rank_ref, enc_ref):
    xin = inp_ref[0]
    scale = jnp.sqrt(_row_sum(xin ** 2))
    x = xin / scale
    e = emb_ref[...]
    m2 = lax.dot_general(x, 2.0 * e, (((1,), (1,)), ((), ())))
    d = (_row_sum(x ** 2) + _row_sum(e ** 2)[:, 0]) - m2
    di = lax.bitcast_convert_type(d, jnp.int32)
    iota_k = lax.broadcasted_iota(jnp.int32, (_S, _K), 1)
    key = lax.shift_left(di - _BASE_I, 10) + iota_k
    kmin = jnp.min(key, axis=1)
    enc = jnp.bitwise_and(kmin, 1023)
    rkey = (kmin - enc) + lax.iota(jnp.int32, _S)
    rkr = rkey[None, :]
    rank = jnp.zeros((1, _S), jnp.int32)
    rkc = rkey[:, None]
    _CH = 64
    for c in range(_S // _CH):
        ri = lax.slice(rkc, (c * _CH, 0), ((c + 1) * _CH, 1))
        rank = rank + jnp.sum((ri < rkr).astype(jnp.int32), axis=0,
                              keepdims=True)
    rank_ref[0, 0] = rank[0]
    enc_ref[0, 0] = enc


def _stage_a(inputs, embedding):
    return pl.pallas_call(
        _stage_a_body,
        grid=(_B,),
        compiler_params=pltpu.CompilerParams(
            dimension_semantics=("parallel",)),
        in_specs=[
            pl.BlockSpec((1, _S, _D), lambda b: (b, 0, 0)),
            pl.BlockSpec((_K, _D), lambda b: (0, 0)),
        ],
        out_specs=[
            pl.BlockSpec((1, 1, _S), lambda b: (b, 0, 0)),
            pl.BlockSpec((1, 1, _S), lambda b: (b, 0, 0)),
        ],
        out_shape=[
            jax.ShapeDtypeStruct((_B, 1, _S), jnp.int32),
            jax.ShapeDtypeStruct((_B, 1, _S), jnp.int32),
        ],
    )(inputs, embedding)


_HALF = _S // 2
_GCH = 96


def _stage_b(embedding, enc0, rank):
    mesh = plsc.VectorSubcoreMesh(core_axis_name="c", subcore_axis_name="s")

    @functools.partial(
        pl.kernel,
        mesh=mesh,
        compiler_params=pltpu.CompilerParams(
            needs_layout_passes=False, use_tc_tiling_on_sc=False),
        out_type=jax.ShapeDtypeStruct((_B * _S, _D), jnp.float32),
        scratch_types=[
            pltpu.VMEM((_S,), jnp.int32),
            pltpu.VMEM((_S,), jnp.int32),
            pltpu.VMEM((_S,), jnp.int32),
            pltpu.VMEM((_GCH, _D), jnp.float32),
            pltpu.SemaphoreType.DMA,
        ],
    )
    def sc_kernel(emb_hbm, enc_hbm, rank_hbm, out_hbm,
                  enc_v, rank_v, fidx_v, rows_v, sem):
        wid = lax.axis_index("s") * 2 + lax.axis_index("c")
        b = wid // 2
        h = wid % 2
        pltpu.sync_copy(enc_hbm, enc_v)
        pltpu.sync_copy(rank_hbm.at[b], rank_v)

        def scatter_body(i, carry):
            sl = pl.ds(i * 16, 16)
            plsc.store_scatter(fidx_v, [rank_v[sl]], enc_v[sl])
            return carry

        lax.fori_loop(0, _S // 16, scatter_body, 0)

        def gather_body(g, carry):
            srow = h * _HALF + g * _GCH
            cp = pltpu.async_copy(
                emb_hbm.at[fidx_v.at[pl.ds(srow, _GCH)]], rows_v, sem)
            cp.wait()
            pltpu.sync_copy(rows_v, out_hbm.at[pl.ds(b * _S + srow, _GCH)])
            return carry

        lax.fori_loop(0, _HALF // _GCH, gather_body, 0)

    return sc_kernel(embedding, enc0, rank)


def _stage_c_body(inp_ref, q_ref, loss_ref):
    xin = inp_ref[...].reshape(_B * _S, _D)
    scale = jnp.sqrt(_row_sum(xin ** 2))
    x = xin / scale
    diff2 = (q_ref[...] - x) ** 2
    m = jnp.mean(diff2)
    loss_ref[...] = ((1.0 + _CC) * m).reshape(1, 1)


def _stage_c(inputs, q):
    return pl.pallas_call(
        _stage_c_body,
        out_shape=jax.ShapeDtypeStruct((1, 1), jnp.float32),
    )(inputs, q)


def kernel(inputs, embedding):
    rank3, enc3 = _stage_a(inputs, embedding)
    enc0 = enc3[0, 0]
    rank = rank3[:, 0, :]
    qflat = _stage_b(embedding, enc0, rank)
    loss = _stage_c(inputs, qflat)
    return qflat.reshape(_B, _S, _D), loss[0, 0]


# speedup vs baseline: 1.3410x; 1.3410x over previous
import functools

import jax
import jax.numpy as jnp
from jax import lax
from jax.experimental import pallas as pl
from jax.experimental.pallas import tpu as pltpu
from jax.experimental.pallas import tpu_sc as plsc

_B, _S, _D, _K = 16, 576, 64, 1024
_CC = 0.99


def _row_sum(sq):
    acc = sq[:, 0:8]
    for r in range(1, 8):
        acc = acc + sq[:, 8 * r:8 * r + 8]
    for w in (4, 2, 1):
        acc = acc[:, :w] + acc[:, w:2 * w]
    return acc


_BASE_I = 0x3F700000


def _stage_a_body(inp_ref, emb_ref, rank_ref, enc_ref, loss_ref,
                  e2_s, E0_s, q2_s, lacc_s):
    b = pl.program_id(0)
    xin = inp_ref[0]
    scale = jnp.sqrt(_row_sum(xin ** 2))
    x = xin / scale
    e = emb_ref[...]

    @pl.when(b == 0)
    def _init_e2():
        e2_s[...] = _row_sum(e ** 2).reshape(1, _K)
        lacc_s[...] = jnp.zeros((1, 1), jnp.float32)

    m2 = lax.dot_general(x, 2.0 * e, (((1,), (1,)), ((), ())))
    x2r = _row_sum(x ** 2)
    d = (x2r + e2_s[...]) - m2
    di = lax.bitcast_convert_type(d, jnp.int32)
    iota_k = lax.broadcasted_iota(jnp.int32, (_S, _K), 1)
    key = lax.shift_left(di - _BASE_I, 10) + iota_k
    kmin = jnp.min(key, axis=1)
    enc = jnp.bitwise_and(kmin, 1023)
    enc_ref[0, 0] = enc

    rkey = (kmin - enc) + lax.iota(jnp.int32, _S)
    cmpf = jnp.where(rkey[:, None] < rkey[None, :], 1.0, 0.0)
    rankf = lax.dot_general(jnp.ones((1, _S), jnp.float32), cmpf,
                            (((1,), (0,)), ((), ())))
    ranki = rankf[0].astype(jnp.int32)
    rank_ref[0, 0] = ranki

    @pl.when(b == 0)
    def _init_e0():
        oh0 = jnp.where(enc[:, None] == lax.broadcasted_iota(
            jnp.int32, (_S, _K), 1), 1.0, 0.0)
        E0 = lax.dot_general(oh0, e, (((1,), (0,)), ((), ())))
        E0_s[...] = E0
        q2_s[...] = jnp.sum(E0 * E0).reshape(1, 1)

    pf = jnp.where(ranki[:, None] == lax.broadcasted_iota(
        jnp.int32, (1, _S), 1), 1.0, 0.0)
    xp = lax.dot_general(pf, x, (((1,), (0,)), ((), ())))
    cross = jnp.sum(E0_s[...] * xp)
    lacc_s[...] = lacc_s[...] + (jnp.sum(x2r) - 2.0 * cross + q2_s[0, 0])
    loss_ref[...] = lacc_s[...] * ((1.0 + _CC) / float(_B * _S * _D))


def _stage_a(inputs, embedding):
    return pl.pallas_call(
        _stage_a_body,
        grid=(_B,),
        in_specs=[
            pl.BlockSpec((1, _S, _D), lambda b: (b, 0, 0)),
            pl.BlockSpec((_K, _D), lambda b: (0, 0)),
        ],
        out_specs=[
            pl.BlockSpec((1, 1, _S), lambda b: (b, 0, 0)),
            pl.BlockSpec((1, 1, _S), lambda b: (b, 0, 0)),
            pl.BlockSpec((1, 1), lambda b: (0, 0)),
        ],
        out_shape=[
            jax.ShapeDtypeStruct((_B, 1, _S), jnp.int32),
            jax.ShapeDtypeStruct((_B, 1, _S), jnp.int32),
            jax.ShapeDtypeStruct((1, 1), jnp.float32),
        ],
        scratch_shapes=[
            pltpu.VMEM((1, _K), jnp.float32),
            pltpu.VMEM((_S, _D), jnp.float32),
            pltpu.VMEM((1, 1), jnp.float32),
            pltpu.VMEM((1, 1), jnp.float32),
        ],
    )(inputs, embedding)


_HALF = _S // 2
_GCH = 96


def _stage_b(embedding, enc0, rank):
    mesh = plsc.VectorSubcoreMesh(core_axis_name="c", subcore_axis_name="s")

    @functools.partial(
        pl.kernel,
        mesh=mesh,
        compiler_params=pltpu.CompilerParams(
            needs_layout_passes=False, use_tc_tiling_on_sc=False),
        out_type=jax.ShapeDtypeStruct((_B * _S, _D), jnp.float32),
        scratch_types=[
            pltpu.VMEM((_S,), jnp.int32),
            pltpu.VMEM((_S,), jnp.int32),
            pltpu.VMEM((_S,), jnp.int32),
            pltpu.VMEM((_GCH, _D), jnp.float32),
            pltpu.SemaphoreType.DMA,
        ],
    )
    def sc_kernel(emb_hbm, enc_hbm, rank_hbm, out_hbm,
                  enc_v, rank_v, fidx_v, rows_v, sem):
        wid = lax.axis_index("s") * 2 + lax.axis_index("c")
        b = wid // 2
        h = wid % 2
        pltpu.sync_copy(enc_hbm, enc_v)
        pltpu.sync_copy(rank_hbm.at[b], rank_v)

        def scatter_body(i, carry):
            sl = pl.ds(i * 16, 16)
            plsc.store_scatter(fidx_v, [rank_v[sl]], enc_v[sl])
            return carry

        lax.fori_loop(0, _S // 16, scatter_body, 0)

        def gather_body(g, carry):
            srow = h * _HALF + g * _GCH
            cp = pltpu.async_copy(
                emb_hbm.at[fidx_v.at[pl.ds(srow, _GCH)]], rows_v, sem)
            cp.wait()
            pltpu.sync_copy(rows_v, out_hbm.at[pl.ds(b * _S + srow, _GCH)])
            return carry

        lax.fori_loop(0, _HALF // _GCH, gather_body, 0)

    return sc_kernel(embedding, enc0, rank)


def kernel(inputs, embedding):
    rank3, enc3, loss = _stage_a(inputs, embedding)
    enc0 = enc3[0, 0]
    rank = rank3[:, 0, :]
    qflat = _stage_b(embedding, enc0, rank)
    return qflat.reshape(_B, _S, _D), loss[0, 0]
